# baseline (speedup 1.0000x reference)
import functools

import jax
import jax.numpy as jnp
from jax import lax
from jax.experimental import pallas as pl
from jax.experimental.pallas import tpu as pltpu
from jax.experimental.pallas import tpu_sc as plsc

N_NODES = 100000
N_EDGES = 6400000
NUM_TYPES = 4

NW = 32
PER_W = N_EDGES // NW
OUTER = 1024
SUB = 128
N_SUB = OUTER // SUB
N_GROUPS = OUTER // 16
N_BLOCKS = -(-PER_W // OUTER)

_MAGIC = 0x5F3759DF


def _rsqrt(x):
    i = plsc.bitcast(x, jnp.int32)
    i = _MAGIC - lax.shift_right_arithmetic(i, 1)
    y = plsc.bitcast(i, jnp.float32)
    hx = 0.5 * x
    for _ in range(2):
        y = y * (1.5 - hx * y * y)
    return y


def _edge_body(packed, ei, recip_hbm, out_len, out_et,
               idxs_a, idxd_a, idxs_b, idxd_b, gs_a, gd_a, gs_b, gd_b,
               len_v, ts_v, td_v, recip_v, sem_a, sem_b):
    pltpu.sync_copy(recip_hbm, recip_v)
    wid = lax.axis_index("s") * 2 + lax.axis_index("c")
    lanes = lax.broadcasted_iota(jnp.int32, (16,), 0)
    cols = [jnp.full((16,), c, jnp.int32) for c in range(4)]

    def base(j):
        return wid * PER_W + jnp.minimum(j * OUTER, PER_W - OUTER)

    def load_idx(bs, idxs, idxd):
        pltpu.sync_copy(ei.at[pl.ds(bs, OUTER)], idxs)
        pltpu.sync_copy(ei.at[pl.ds(N_EDGES + bs, OUTER)], idxd)

    def fire(idxs, idxd, gs, gd, sem):
        for s in range(N_SUB):
            o = s * SUB
            pltpu.make_async_copy(packed.at[idxs.at[pl.ds(o, SUB)]],
                                  gs.at[pl.ds(o, SUB)], sem).start()
            pltpu.make_async_copy(packed.at[idxd.at[pl.ds(o, SUB)]],
                                  gd.at[pl.ds(o, SUB)], sem).start()

    def drain(idxs, idxd, gs, gd, sem):
        for s in range(N_SUB):
            o = s * SUB
            pltpu.make_async_copy(packed.at[idxs.at[pl.ds(o, SUB)]],
                                  gs.at[pl.ds(o, SUB)], sem).wait()
            pltpu.make_async_copy(packed.at[idxd.at[pl.ds(o, SUB)]],
                                  gd.at[pl.ds(o, SUB)], sem).wait()

    def compute(gs_v, gd_v):
        def group4(q, c):
            for u in range(4):
                g = q * 4 + u
                _one_group(gs_v, gd_v, g)
            return c

        def _one_group(gs_v, gd_v, g):
            row = lanes + g * 16
            xs = plsc.bitcast(plsc.load_gather(gs_v, [row, cols[0]]),
                              jnp.float32)
            ys = plsc.bitcast(plsc.load_gather(gs_v, [row, cols[1]]),
                              jnp.float32)
            zs = plsc.bitcast(plsc.load_gather(gs_v, [row, cols[2]]),
                              jnp.float32)
            tsi = plsc.load_gather(gs_v, [row, cols[3]])
            xd = plsc.bitcast(plsc.load_gather(gd_v, [row, cols[0]]),
                              jnp.float32)
            yd = plsc.bitcast(plsc.load_gather(gd_v, [row, cols[1]]),
                              jnp.float32)
            zd = plsc.bitcast(plsc.load_gather(gd_v, [row, cols[2]]),
                              jnp.float32)
            tdi = plsc.load_gather(gd_v, [row, cols[3]])
            dx = xd - xs
            dy = yd - ys
            dz = zd - zs
            ss = dx * dx + dy * dy + dz * dz
            r = ss * _rsqrt(ss)
            et = tsi * NUM_TYPES + tdi
            rc = plsc.load_gather(recip_v, [et])
            off = g * 16
            len_v[pl.ds(off, 16)] = r * rc
            ts_v[pl.ds(off, 16)] = tsi
            td_v[pl.ds(off, 16)] = tdi

        lax.fori_loop(0, N_GROUPS // 4, group4, 0)

    def flush(bs):
        pltpu.sync_copy(len_v, out_len.at[pl.ds(bs, OUTER)])
        pltpu.sync_copy(ts_v, out_et.at[pl.ds(bs, OUTER)])
        pltpu.sync_copy(td_v, out_et.at[pl.ds(N_EDGES + bs, OUTER)])

    load_idx(base(0), idxs_a, idxd_a)
    fire(idxs_a, idxd_a, gs_a, gd_a, sem_a)

    def body(jj, c):
        j = jj * 2
        load_idx(base(j + 1), idxs_b, idxd_b)
        fire(idxs_b, idxd_b, gs_b, gd_b, sem_b)
        drain(idxs_a, idxd_a, gs_a, gd_a, sem_a)
        compute(gs_a, gd_a)
        flush(base(j))
        load_idx(base(j + 2), idxs_a, idxd_a)
        fire(idxs_a, idxd_a, gs_a, gd_a, sem_a)
        drain(idxs_b, idxd_b, gs_b, gd_b, sem_b)
        compute(gs_b, gd_b)
        flush(base(j + 1))
        return c

    lax.fori_loop(0, N_BLOCKS // 2, body, 0)
    drain(idxs_a, idxd_a, gs_a, gd_a, sem_a)


_edge_call = functools.partial(
    pl.kernel,
    out_type=[
        jax.ShapeDtypeStruct((N_EDGES,), jnp.float32),
        jax.ShapeDtypeStruct((2 * N_EDGES,), jnp.int32),
    ],
    mesh=plsc.VectorSubcoreMesh(core_axis_name="c", subcore_axis_name="s"),
    compiler_params=pltpu.CompilerParams(
        needs_layout_passes=False, use_tc_tiling_on_sc=False),
    scratch_types=[
        pltpu.VMEM((OUTER,), jnp.int32),
        pltpu.VMEM((OUTER,), jnp.int32),
        pltpu.VMEM((OUTER,), jnp.int32),
        pltpu.VMEM((OUTER,), jnp.int32),
        pltpu.VMEM((OUTER, 16), jnp.int32),
        pltpu.VMEM((OUTER, 16), jnp.int32),
        pltpu.VMEM((OUTER, 16), jnp.int32),
        pltpu.VMEM((OUTER, 16), jnp.int32),
        pltpu.VMEM((OUTER,), jnp.float32),
        pltpu.VMEM((OUTER,), jnp.int32),
        pltpu.VMEM((OUTER,), jnp.int32),
        pltpu.VMEM((16,), jnp.float32),
        pltpu.SemaphoreType.DMA,
        pltpu.SemaphoreType.DMA,
    ],
)(_edge_body)


def kernel(pos, edge_index, atom_type, rmax_recip):
    pos_bits = lax.bitcast_convert_type(pos.astype(jnp.float32), jnp.int32)
    pad = jnp.zeros((N_NODES, 12), jnp.int32)
    packed = jnp.concatenate(
        [pos_bits, atom_type.astype(jnp.int32).reshape(-1, 1), pad], axis=1)
    ei_flat = edge_index.astype(jnp.int32).reshape(-1)
    out_len, out_et = _edge_call(packed, ei_flat,
                                 rmax_recip.astype(jnp.float32))
    return out_len.reshape(-1, 1), out_et.reshape(2, -1)


# speedup vs baseline: 1.0300x; 1.0300x over previous
import functools

import jax
import jax.numpy as jnp
from jax import lax
from jax.experimental import pallas as pl
from jax.experimental.pallas import tpu as pltpu
from jax.experimental.pallas import tpu_sc as plsc

N_NODES = 100000
N_EDGES = 6400000
NUM_TYPES = 4

NW = 32
PER_W = N_EDGES // NW
OUTER = 1024
SUB = 128
N_SUB = OUTER // SUB
N_GROUPS = OUTER // 16
N_BLOCKS = -(-PER_W // OUTER)

_MAGIC = 0x5F3759DF


def _rsqrt(x):
    i = plsc.bitcast(x, jnp.int32)
    i = _MAGIC - lax.shift_right_arithmetic(i, 1)
    y = plsc.bitcast(i, jnp.float32)
    hx = 0.5 * x
    for _ in range(2):
        y = y * (1.5 - hx * y * y)
    return y


def _edge_body(packed, ei, recip_hbm, out_len, out_et,
               idxs_a, idxd_a, idxs_b, idxd_b, gs_a, gd_a, gs_b, gd_b,
               len_v, ts_v, td_v, recip_v, sem_a, sem_b):
    pltpu.sync_copy(recip_hbm, recip_v)
    wid = lax.axis_index("s") * 2 + lax.axis_index("c")
    lanes = lax.broadcasted_iota(jnp.int32, (16,), 0)
    cols = [jnp.full((16,), c, jnp.int32) for c in range(4)]

    def base(j):
        return wid * PER_W + jnp.minimum(j * OUTER, PER_W - OUTER)

    def load_idx(bs, idxs, idxd):
        pltpu.sync_copy(ei.at[pl.ds(bs, OUTER)], idxs)
        pltpu.sync_copy(ei.at[pl.ds(N_EDGES + bs, OUTER)], idxd)

    def fire(idxs, idxd, gs, gd, sem):
        for s in range(N_SUB):
            o = s * SUB
            pltpu.make_async_copy(packed.at[idxs.at[pl.ds(o, SUB)]],
                                  gs.at[pl.ds(o, SUB)], sem).start()
            pltpu.make_async_copy(packed.at[idxd.at[pl.ds(o, SUB)]],
                                  gd.at[pl.ds(o, SUB)], sem).start()

    def drain(idxs, idxd, gs, gd, sem):
        for s in range(N_SUB):
            o = s * SUB
            pltpu.make_async_copy(packed.at[idxs.at[pl.ds(o, SUB)]],
                                  gs.at[pl.ds(o, SUB)], sem).wait()
            pltpu.make_async_copy(packed.at[idxd.at[pl.ds(o, SUB)]],
                                  gd.at[pl.ds(o, SUB)], sem).wait()

    def compute(gs_v, gd_v):
        def group(g, c):
            row = lanes + g * 16
            xs = plsc.bitcast(plsc.load_gather(gs_v, [row, cols[0]]),
                              jnp.float32)
            ys = plsc.bitcast(plsc.load_gather(gs_v, [row, cols[1]]),
                              jnp.float32)
            zs = plsc.bitcast(plsc.load_gather(gs_v, [row, cols[2]]),
                              jnp.float32)
            tsi = plsc.load_gather(gs_v, [row, cols[3]])
            xd = plsc.bitcast(plsc.load_gather(gd_v, [row, cols[0]]),
                              jnp.float32)
            yd = plsc.bitcast(plsc.load_gather(gd_v, [row, cols[1]]),
                              jnp.float32)
            zd = plsc.bitcast(plsc.load_gather(gd_v, [row, cols[2]]),
                              jnp.float32)
            tdi = plsc.load_gather(gd_v, [row, cols[3]])
            dx = xd - xs
            dy = yd - ys
            dz = zd - zs
            ss = dx * dx + dy * dy + dz * dz
            r = ss * _rsqrt(ss)
            et = tsi * NUM_TYPES + tdi
            rc = plsc.load_gather(recip_v, [et])
            off = g * 16
            len_v[pl.ds(off, 16)] = r * rc
            ts_v[pl.ds(off, 16)] = tsi
            td_v[pl.ds(off, 16)] = tdi
            return c

        lax.fori_loop(0, N_GROUPS, group, 0)

    def flush(bs):
        pltpu.sync_copy(len_v, out_len.at[pl.ds(bs, OUTER)])
        pltpu.sync_copy(ts_v, out_et.at[pl.ds(bs, OUTER)])
        pltpu.sync_copy(td_v, out_et.at[pl.ds(N_EDGES + bs, OUTER)])

    load_idx(base(0), idxs_a, idxd_a)
    fire(idxs_a, idxd_a, gs_a, gd_a, sem_a)

    def body(jj, c):
        j = jj * 2
        load_idx(base(j + 1), idxs_b, idxd_b)
        fire(idxs_b, idxd_b, gs_b, gd_b, sem_b)
        drain(idxs_a, idxd_a, gs_a, gd_a, sem_a)
        compute(gs_a, gd_a)
        flush(base(j))
        load_idx(base(j + 2), idxs_a, idxd_a)
        fire(idxs_a, idxd_a, gs_a, gd_a, sem_a)
        drain(idxs_b, idxd_b, gs_b, gd_b, sem_b)
        compute(gs_b, gd_b)
        flush(base(j + 1))
        return c

    lax.fori_loop(0, N_BLOCKS // 2, body, 0)
    drain(idxs_a, idxd_a, gs_a, gd_a, sem_a)


_edge_call = functools.partial(
    pl.kernel,
    out_type=[
        jax.ShapeDtypeStruct((N_EDGES,), jnp.float32),
        jax.ShapeDtypeStruct((2 * N_EDGES,), jnp.int32),
    ],
    mesh=plsc.VectorSubcoreMesh(core_axis_name="c", subcore_axis_name="s"),
    compiler_params=pltpu.CompilerParams(
        needs_layout_passes=False, use_tc_tiling_on_sc=False),
    scratch_types=[
        pltpu.VMEM((OUTER,), jnp.int32),
        pltpu.VMEM((OUTER,), jnp.int32),
        pltpu.VMEM((OUTER,), jnp.int32),
        pltpu.VMEM((OUTER,), jnp.int32),
        pltpu.VMEM((OUTER, 16), jnp.int32),
        pltpu.VMEM((OUTER, 16), jnp.int32),
        pltpu.VMEM((OUTER, 16), jnp.int32),
        pltpu.VMEM((OUTER, 16), jnp.int32),
        pltpu.VMEM((OUTER,), jnp.float32),
        pltpu.VMEM((OUTER,), jnp.int32),
        pltpu.VMEM((OUTER,), jnp.int32),
        pltpu.VMEM((16,), jnp.float32),
        pltpu.SemaphoreType.DMA,
        pltpu.SemaphoreType.DMA,
    ],
)(_edge_body)


def kernel(pos, edge_index, atom_type, rmax_recip):
    pos_bits = lax.bitcast_convert_type(pos.astype(jnp.float32), jnp.int32)
    pad = jnp.zeros((N_NODES, 12), jnp.int32)
    packed = jnp.concatenate(
        [pos_bits, atom_type.astype(jnp.int32).reshape(-1, 1), pad], axis=1)
    ei_flat = edge_index.astype(jnp.int32).reshape(-1)
    out_len, out_et = _edge_call(packed, ei_flat,
                                 rmax_recip.astype(jnp.float32))
    return out_len.reshape(-1, 1), out_et.reshape(2, -1)


# speedup vs baseline: 1.1088x; 1.0764x over previous
import functools

import jax
import jax.numpy as jnp
from jax import lax
from jax.experimental import pallas as pl
from jax.experimental.pallas import tpu as pltpu
from jax.experimental.pallas import tpu_sc as plsc

N_NODES = 100000
N_EDGES = 6400000
NUM_TYPES = 4

NW = 32
PER_W = N_EDGES // NW
OUTER = 1024
SUB = 128
N_SUB = OUTER // SUB
N_GROUPS = OUTER // 16
N_BLOCKS = -(-PER_W // OUTER)

_MAGIC = 0x5F3759DF


def _rsqrt(x):
    i = plsc.bitcast(x, jnp.int32)
    i = _MAGIC - lax.shift_right_arithmetic(i, 1)
    y = plsc.bitcast(i, jnp.float32)
    hx = 0.5 * x
    for _ in range(2):
        y = y * (1.5 - hx * y * y)
    return y


def _edge_body(packed, ei, recip_hbm, out_len, out_et,
               idxs_a, idxd_a, idxs_b, idxd_b, gs_a, gd_a, gs_b, gd_b,
               len_v, ts_v, td_v, recip_v, sem_a, sem_b):
    pltpu.sync_copy(recip_hbm, recip_v)
    wid = lax.axis_index("s") * 2 + lax.axis_index("c")
    lanes = lax.broadcasted_iota(jnp.int32, (16,), 0)
    cols = [jnp.full((16,), c, jnp.int32) for c in range(4)]

    def base(j):
        return wid * PER_W + jnp.minimum(j * OUTER, PER_W - OUTER)

    def load_idx(bs, idxs, idxd):
        pltpu.sync_copy(ei.at[pl.ds(bs, OUTER)], idxs)
        pltpu.sync_copy(ei.at[pl.ds(N_EDGES + bs, OUTER)], idxd)

    def fire(idxs, idxd, gs, gd, sem):
        for s in range(N_SUB):
            o = s * SUB
            pltpu.make_async_copy(packed.at[idxs.at[pl.ds(o, SUB)]],
                                  gs.at[pl.ds(o, SUB)], sem).start()
            pltpu.make_async_copy(packed.at[idxd.at[pl.ds(o, SUB)]],
                                  gd.at[pl.ds(o, SUB)], sem).start()

    def drain(idxs, idxd, gs, gd, sem):
        for s in range(N_SUB):
            o = s * SUB
            pltpu.make_async_copy(packed.at[idxs.at[pl.ds(o, SUB)]],
                                  gs.at[pl.ds(o, SUB)], sem).wait()
            pltpu.make_async_copy(packed.at[idxd.at[pl.ds(o, SUB)]],
                                  gd.at[pl.ds(o, SUB)], sem).wait()

    def compute(gs_v, gd_v):
        def group(g, c):
            row = lanes + g * 16
            xs = plsc.bitcast(plsc.load_gather(gs_v, [row, cols[0]]),
                              jnp.float32)
            ys = plsc.bitcast(plsc.load_gather(gs_v, [row, cols[1]]),
                              jnp.float32)
            zs = plsc.bitcast(plsc.load_gather(gs_v, [row, cols[2]]),
                              jnp.float32)
            tsi = plsc.load_gather(gs_v, [row, cols[3]])
            xd = plsc.bitcast(plsc.load_gather(gd_v, [row, cols[0]]),
                              jnp.float32)
            yd = plsc.bitcast(plsc.load_gather(gd_v, [row, cols[1]]),
                              jnp.float32)
            zd = plsc.bitcast(plsc.load_gather(gd_v, [row, cols[2]]),
                              jnp.float32)
            tdi = plsc.load_gather(gd_v, [row, cols[3]])
            dx = xd - xs
            dy = yd - ys
            dz = zd - zs
            ss = dx * dx + dy * dy + dz * dz
            r = ss * _rsqrt(ss)
            et = tsi * NUM_TYPES + tdi
            rc = plsc.load_gather(recip_v, [et])
            off = g * 16
            len_v[pl.ds(off, 16)] = r * rc
            ts_v[pl.ds(off, 16)] = tsi
            td_v[pl.ds(off, 16)] = tdi
            return c

        lax.fori_loop(0, N_GROUPS, group, 0)

    def flush(bs):
        pltpu.sync_copy(len_v, out_len.at[pl.ds(bs, OUTER)])
        pltpu.sync_copy(ts_v, out_et.at[pl.ds(bs, OUTER)])
        pltpu.sync_copy(td_v, out_et.at[pl.ds(N_EDGES + bs, OUTER)])

    load_idx(base(0), idxs_a, idxd_a)
    fire(idxs_a, idxd_a, gs_a, gd_a, sem_a)

    def body(jj, c):
        j = jj * 2
        load_idx(base(j + 1), idxs_b, idxd_b)
        fire(idxs_b, idxd_b, gs_b, gd_b, sem_b)
        drain(idxs_a, idxd_a, gs_a, gd_a, sem_a)
        compute(gs_a, gd_a)
        flush(base(j))
        load_idx(base(j + 2), idxs_a, idxd_a)
        fire(idxs_a, idxd_a, gs_a, gd_a, sem_a)
        drain(idxs_b, idxd_b, gs_b, gd_b, sem_b)
        compute(gs_b, gd_b)
        flush(base(j + 1))
        return c

    lax.fori_loop(0, N_BLOCKS // 2, body, 0)
    drain(idxs_a, idxd_a, gs_a, gd_a, sem_a)


_edge_call = functools.partial(
    pl.kernel,
    out_type=[
        jax.ShapeDtypeStruct((N_EDGES,), jnp.float32),
        jax.ShapeDtypeStruct((2 * N_EDGES,), jnp.int32),
    ],
    mesh=plsc.VectorSubcoreMesh(core_axis_name="c", subcore_axis_name="s"),
    compiler_params=pltpu.CompilerParams(
        needs_layout_passes=False, use_tc_tiling_on_sc=False),
    scratch_types=[
        pltpu.VMEM((OUTER,), jnp.int32),
        pltpu.VMEM((OUTER,), jnp.int32),
        pltpu.VMEM((OUTER,), jnp.int32),
        pltpu.VMEM((OUTER,), jnp.int32),
        pltpu.VMEM((OUTER, 8), jnp.int32),
        pltpu.VMEM((OUTER, 8), jnp.int32),
        pltpu.VMEM((OUTER, 8), jnp.int32),
        pltpu.VMEM((OUTER, 8), jnp.int32),
        pltpu.VMEM((OUTER,), jnp.float32),
        pltpu.VMEM((OUTER,), jnp.int32),
        pltpu.VMEM((OUTER,), jnp.int32),
        pltpu.VMEM((16,), jnp.float32),
        pltpu.SemaphoreType.DMA,
        pltpu.SemaphoreType.DMA,
    ],
)(_edge_body)


def kernel(pos, edge_index, atom_type, rmax_recip):
    pos_bits = lax.bitcast_convert_type(pos.astype(jnp.float32), jnp.int32)
    pad = jnp.zeros((N_NODES, 4), jnp.int32)
    packed = jnp.concatenate(
        [pos_bits, atom_type.astype(jnp.int32).reshape(-1, 1), pad], axis=1)
    ei_flat = edge_index.astype(jnp.int32).reshape(-1)
    out_len, out_et = _edge_call(packed, ei_flat,
                                 rmax_recip.astype(jnp.float32))
    return out_len.reshape(-1, 1), out_et.reshape(2, -1)


# speedup vs baseline: 1.1827x; 1.0667x over previous
import functools

import jax
import jax.numpy as jnp
from jax import lax
from jax.experimental import pallas as pl
from jax.experimental.pallas import tpu as pltpu
from jax.experimental.pallas import tpu_sc as plsc

N_NODES = 100000
N_EDGES = 6400000
NUM_TYPES = 4

NW = 32
PER_W = N_EDGES // NW
OUTER = 2048
SUB = 128
N_SUB = OUTER // SUB
N_GROUPS = OUTER // 16
N_BLOCKS = -(-PER_W // OUTER)

_MAGIC = 0x5F3759DF


def _rsqrt(x):
    i = plsc.bitcast(x, jnp.int32)
    i = _MAGIC - lax.shift_right_arithmetic(i, 1)
    y = plsc.bitcast(i, jnp.float32)
    hx = 0.5 * x
    for _ in range(2):
        y = y * (1.5 - hx * y * y)
    return y


def _edge_body(packed, ei, recip_hbm, out_len, out_et,
               idxs_a, idxd_a, idxs_b, idxd_b, gs_a, gd_a, gs_b, gd_b,
               len_v, ts_v, td_v, recip_v, sem_a, sem_b):
    pltpu.sync_copy(recip_hbm, recip_v)
    wid = lax.axis_index("s") * 2 + lax.axis_index("c")
    lanes = lax.broadcasted_iota(jnp.int32, (16,), 0)
    cols = [jnp.full((16,), c, jnp.int32) for c in range(4)]

    def base(j):
        return wid * PER_W + jnp.minimum(j * OUTER, PER_W - OUTER)

    def load_idx(bs, idxs, idxd):
        pltpu.sync_copy(ei.at[pl.ds(bs, OUTER)], idxs)
        pltpu.sync_copy(ei.at[pl.ds(N_EDGES + bs, OUTER)], idxd)

    def fire(idxs, idxd, gs, gd, sem):
        for s in range(N_SUB):
            o = s * SUB
            pltpu.make_async_copy(packed.at[idxs.at[pl.ds(o, SUB)]],
                                  gs.at[pl.ds(o, SUB)], sem).start()
            pltpu.make_async_copy(packed.at[idxd.at[pl.ds(o, SUB)]],
                                  gd.at[pl.ds(o, SUB)], sem).start()

    def drain(idxs, idxd, gs, gd, sem):
        for s in range(N_SUB):
            o = s * SUB
            pltpu.make_async_copy(packed.at[idxs.at[pl.ds(o, SUB)]],
                                  gs.at[pl.ds(o, SUB)], sem).wait()
            pltpu.make_async_copy(packed.at[idxd.at[pl.ds(o, SUB)]],
                                  gd.at[pl.ds(o, SUB)], sem).wait()

    def compute(gs_v, gd_v):
        def group(g, c):
            row = lanes + g * 16
            xs = plsc.bitcast(plsc.load_gather(gs_v, [row, cols[0]]),
                              jnp.float32)
            ys = plsc.bitcast(plsc.load_gather(gs_v, [row, cols[1]]),
                              jnp.float32)
            zs = plsc.bitcast(plsc.load_gather(gs_v, [row, cols[2]]),
                              jnp.float32)
            tsi = plsc.load_gather(gs_v, [row, cols[3]])
            xd = plsc.bitcast(plsc.load_gather(gd_v, [row, cols[0]]),
                              jnp.float32)
            yd = plsc.bitcast(plsc.load_gather(gd_v, [row, cols[1]]),
                              jnp.float32)
            zd = plsc.bitcast(plsc.load_gather(gd_v, [row, cols[2]]),
                              jnp.float32)
            tdi = plsc.load_gather(gd_v, [row, cols[3]])
            dx = xd - xs
            dy = yd - ys
            dz = zd - zs
            ss = dx * dx + dy * dy + dz * dz
            r = ss * _rsqrt(ss)
            et = tsi * NUM_TYPES + tdi
            rc = plsc.load_gather(recip_v, [et])
            off = g * 16
            len_v[pl.ds(off, 16)] = r * rc
            ts_v[pl.ds(off, 16)] = tsi
            td_v[pl.ds(off, 16)] = tdi
            return c

        lax.fori_loop(0, N_GROUPS, group, 0)

    def flush(bs):
        pltpu.sync_copy(len_v, out_len.at[pl.ds(bs, OUTER)])
        pltpu.sync_copy(ts_v, out_et.at[pl.ds(bs, OUTER)])
        pltpu.sync_copy(td_v, out_et.at[pl.ds(N_EDGES + bs, OUTER)])

    load_idx(base(0), idxs_a, idxd_a)
    fire(idxs_a, idxd_a, gs_a, gd_a, sem_a)

    def body(jj, c):
        j = jj * 2
        load_idx(base(j + 1), idxs_b, idxd_b)
        fire(idxs_b, idxd_b, gs_b, gd_b, sem_b)
        drain(idxs_a, idxd_a, gs_a, gd_a, sem_a)
        compute(gs_a, gd_a)
        flush(base(j))
        load_idx(base(j + 2), idxs_a, idxd_a)
        fire(idxs_a, idxd_a, gs_a, gd_a, sem_a)
        drain(idxs_b, idxd_b, gs_b, gd_b, sem_b)
        compute(gs_b, gd_b)
        flush(base(j + 1))
        return c

    lax.fori_loop(0, N_BLOCKS // 2, body, 0)
    drain(idxs_a, idxd_a, gs_a, gd_a, sem_a)


_edge_call = functools.partial(
    pl.kernel,
    out_type=[
        jax.ShapeDtypeStruct((N_EDGES,), jnp.float32),
        jax.ShapeDtypeStruct((2 * N_EDGES,), jnp.int32),
    ],
    mesh=plsc.VectorSubcoreMesh(core_axis_name="c", subcore_axis_name="s"),
    compiler_params=pltpu.CompilerParams(
        needs_layout_passes=False, use_tc_tiling_on_sc=False),
    scratch_types=[
        pltpu.VMEM((OUTER,), jnp.int32),
        pltpu.VMEM((OUTER,), jnp.int32),
        pltpu.VMEM((OUTER,), jnp.int32),
        pltpu.VMEM((OUTER,), jnp.int32),
        pltpu.VMEM((OUTER, 8), jnp.int32),
        pltpu.VMEM((OUTER, 8), jnp.int32),
        pltpu.VMEM((OUTER, 8), jnp.int32),
        pltpu.VMEM((OUTER, 8), jnp.int32),
        pltpu.VMEM((OUTER,), jnp.float32),
        pltpu.VMEM((OUTER,), jnp.int32),
        pltpu.VMEM((OUTER,), jnp.int32),
        pltpu.VMEM((16,), jnp.float32),
        pltpu.SemaphoreType.DMA,
        pltpu.SemaphoreType.DMA,
    ],
)(_edge_body)


def kernel(pos, edge_index, atom_type, rmax_recip):
    pos_bits = lax.bitcast_convert_type(pos.astype(jnp.float32), jnp.int32)
    pad = jnp.zeros((N_NODES, 4), jnp.int32)
    packed = jnp.concatenate(
        [pos_bits, atom_type.astype(jnp.int32).reshape(-1, 1), pad], axis=1)
    ei_flat = edge_index.astype(jnp.int32).reshape(-1)
    out_len, out_et = _edge_call(packed, ei_flat,
                                 rmax_recip.astype(jnp.float32))
    return out_len.reshape(-1, 1), out_et.reshape(2, -1)
